# Initial kernel scaffold; baseline (speedup 1.0000x reference)
import jax
import jax.numpy as jnp
from jax.experimental import pallas as pl


def kernel(x, edge_index, edge_weight, W1, b1, ln1_g, ln1_b, W2, b2, ln2_g, ln2_b, Wp, bp, lnp_g, lnp_b):
    raise NotImplementedError("write your pallas kernel here")


# speedup vs baseline: 7.7054x; 7.7054x over previous
import functools

import jax
import jax.numpy as jnp
from jax import lax
from jax.experimental import pallas as pl
from jax.experimental.pallas import tpu as pltpu
from jax.experimental.pallas import tpu_sc as plsc

N = 10000
E = 320000
M = 10000
IN_DIM, HID, OUT_DIM, PROJ = 128, 256, 128, 128

NC, NS = 2, 16
NW = NC * NS
EPW = E // NW
WIN = 80
NWIN_W = EPW // WIN
RPT = M // NS
VPT = N // 10

_MESH = plsc.VectorSubcoreMesh(core_axis_name="c", subcore_axis_name="s")


def _wid():
    return lax.axis_index("s") * NC + lax.axis_index("c")


def _sc_pass_body(with_degrees, *refs):
    if with_degrees:
        (src, idx_s_hbm, idx_d_hbm, ew_hbm, zrows, out, dd_out, bd_out,
         idx_s, idx_d, rows, w_v, one_v, zv_v, acc, dd_sh, bd_sh) = refs
    else:
        (src, idx_s_hbm, idx_d_hbm, zrows, out,
         idx_s, idx_d, rows, acc) = refs
    c = lax.axis_index("c")
    s = lax.axis_index("s")
    w = _wid()

    @pl.when(s < 10)
    def _zero_acc():
        pltpu.sync_copy(zrows.at[pl.ds(s * VPT, VPT)],
                        acc.at[pl.ds(s * VPT, VPT)])
    pltpu.sync_copy(idx_s_hbm.at[w], idx_s)
    pltpu.sync_copy(idx_d_hbm.at[w], idx_d)
    if with_degrees:
        def _z(j, carry):
            zv_v[pl.ds(j * 16, 16)] = jnp.zeros((16,), jnp.float32)
            return carry
        lax.fori_loop(0, 63, _z, 0)
        @pl.when(s < 10)
        def _zero_deg():
            pltpu.sync_copy(zv_v.at[pl.ds(0, VPT)],
                            dd_sh.at[pl.ds(s * VPT, VPT)])
            pltpu.sync_copy(zv_v.at[pl.ds(0, VPT)],
                            bd_sh.at[pl.ds(s * VPT, VPT)])
        for j in range(WIN // 16):
            one_v[pl.ds(j * 16, 16)] = jnp.full((16,), 1.0, jnp.float32)
    plsc.subcore_barrier()

    def body(i, carry):
        pltpu.sync_copy(src.at[idx_s.at[i]], rows)
        pltpu.sync_copy(rows, acc.at[idx_d.at[i]], add=True)
        if with_degrees:
            pltpu.sync_copy(ew_hbm.at[idx_d.at[i]], w_v)
            pltpu.sync_copy(w_v, dd_sh.at[idx_s.at[i]], add=True)
            pltpu.sync_copy(one_v, bd_sh.at[idx_d.at[i]], add=True)
        return carry

    lax.fori_loop(0, NWIN_W, body, 0)
    plsc.subcore_barrier()
    @pl.when(s < 10)
    def _out_acc():
        pltpu.sync_copy(acc.at[pl.ds(s * VPT, VPT)],
                        out.at[c, pl.ds(s * VPT, VPT)])
    if with_degrees:
        @pl.when(s == 0)
        def _out_deg():
            pltpu.sync_copy(dd_sh, dd_out.at[c, 0])
            pltpu.sync_copy(bd_sh, bd_out.at[c, 0])


def _make_sc_pass(with_degrees):
    out_type = [jax.ShapeDtypeStruct((NC, M, 128), jnp.float32)]
    scratch = [
        pltpu.VMEM((NWIN_W, WIN), jnp.int32),
        pltpu.VMEM((NWIN_W, WIN), jnp.int32),
        pltpu.VMEM((WIN, 128), jnp.float32),
    ]
    if with_degrees:
        out_type += [jax.ShapeDtypeStruct((NC, 1, N), jnp.float32),
                     jax.ShapeDtypeStruct((NC, 1, M), jnp.float32)]
        scratch += [
            pltpu.VMEM((WIN,), jnp.float32),
            pltpu.VMEM((WIN,), jnp.float32),
            pltpu.VMEM((1008,), jnp.float32),
        ]
    scratch += [pltpu.VMEM_SHARED((M, 128), jnp.float32)]
    if with_degrees:
        scratch += [pltpu.VMEM_SHARED((N,), jnp.float32),
                    pltpu.VMEM_SHARED((M,), jnp.float32)]
    return pl.kernel(
        functools.partial(_sc_pass_body, with_degrees),
        out_type=out_type, mesh=_MESH, scratch_types=scratch,
        name="sc_pass_deg" if with_degrees else "sc_pass")


_sc_pass_deg = _make_sc_pass(True)
_sc_pass = _make_sc_pass(False)


_BR = 1000
_GRID = M // _BR


def _dot(a, b):
    return lax.dot_general(a, b, (((1,), (0,)), ((), ())),
                           preferred_element_type=jnp.float32,
                           precision=lax.Precision.HIGHEST)


def _ln(h, g, b, eps=1e-5):
    mu = jnp.mean(h, axis=-1, keepdims=True)
    var = jnp.mean((h - mu) ** 2, axis=-1, keepdims=True)
    return (h - mu) * lax.rsqrt(var + eps) * g + b


def _tc_combine_body(a0, a1, bd0, bd1, ew, out_a, out_bn):
    bd = bd0[...] + bd1[...]
    binv = jnp.where(bd > 0, 1.0 / jnp.where(bd > 0, bd, 1.0), 0.0)
    bn = binv * ew[...]
    out_bn[...] = bn
    out_a[...] = (a0[...] + a1[...]) * bn


def _row_spec(w):
    return pl.BlockSpec((_BR, w), lambda i: (i, 0))


def _full_spec(r, c):
    return pl.BlockSpec((r, c), lambda i: (0, 0))


_tc_combine = pl.pallas_call(
    _tc_combine_body,
    grid=(_GRID,),
    in_specs=[_row_spec(128), _row_spec(128), _row_spec(1), _row_spec(1),
              _row_spec(1)],
    out_specs=[_row_spec(128), _row_spec(1)],
    out_shape=[jax.ShapeDtypeStruct((M, 128), jnp.float32),
               jax.ShapeDtypeStruct((M, 1), jnp.float32)],
)


def _tc_combine2_body(a0, a1, bn, out_a):
    out_a[...] = (a0[...] + a1[...]) * bn[...]


_tc_combine2 = pl.pallas_call(
    _tc_combine2_body,
    grid=(_GRID,),
    in_specs=[_row_spec(128), _row_spec(128), _row_spec(1)],
    out_specs=_row_spec(128),
    out_shape=jax.ShapeDtypeStruct((M, 128), jnp.float32),
)


def _tc_mid_body(p0, p1, dd0, dd1, W1, b1, g1, bb1, W2, out_t2, out_dinv):
    dd = dd0[...] + dd1[...]
    dinv = jnp.where(dd > 0, 1.0 / jnp.where(dd > 0, dd, 1.0), 0.0)
    out_dinv[...] = dinv
    p = (p0[...] + p1[...]) * dinv
    h = _dot(p, W1[...]) + b1[...]
    h = jnp.maximum(_ln(h, g1[...], bb1[...]), 0.0)
    out_t2[...] = _dot(h, W2[...])


_tc_mid = pl.pallas_call(
    _tc_mid_body,
    grid=(_GRID,),
    in_specs=[_row_spec(128), _row_spec(128), _row_spec(1), _row_spec(1),
              _full_spec(128, HID), _full_spec(1, HID), _full_spec(1, HID),
              _full_spec(1, HID), _full_spec(HID, 128)],
    out_specs=[_row_spec(128), _row_spec(1)],
    out_shape=[jax.ShapeDtypeStruct((N, 128), jnp.float32),
               jax.ShapeDtypeStruct((N, 1), jnp.float32)],
)


def _tc_final_body(p0, p1, dinv, b2, g2, bb2, Wp, bp, gp, bbp, out):
    p = (p0[...] + p1[...]) * dinv[...] + b2[...]
    h = _ln(p, g2[...], bb2[...])
    h = jnp.maximum(_dot(h, Wp[...]) + bp[...], 0.0)
    out[...] = _ln(h, gp[...], bbp[...])


_tc_final = pl.pallas_call(
    _tc_final_body,
    grid=(_GRID,),
    in_specs=[_row_spec(128), _row_spec(128), _row_spec(1),
              _full_spec(1, 128), _full_spec(1, 128), _full_spec(1, 128),
              _full_spec(128, 128), _full_spec(1, 128), _full_spec(1, 128),
              _full_spec(1, 128)],
    out_specs=_row_spec(128),
    out_shape=jax.ShapeDtypeStruct((N, PROJ), jnp.float32),
)


def kernel(x, edge_index, edge_weight, W1, b1, ln1_g, ln1_b, W2, b2, ln2_g,
           ln2_b, Wp, bp, lnp_g, lnp_b):
    node_idx = edge_index[0].reshape(NW, NWIN_W, WIN)
    edge_idx = edge_index[1].reshape(NW, NWIN_W, WIN)
    zrows = jnp.zeros((M, 128), jnp.float32)

    a1p, ddp, bdp = _sc_pass_deg(x, node_idx, edge_idx, edge_weight, zrows)
    r2 = lambda v: v.reshape(-1, 1)
    a1s, bnorm = _tc_combine(a1p[0], a1p[1], r2(bdp[0]), r2(bdp[1]),
                             r2(edge_weight))
    p1p, = _sc_pass(a1s, edge_idx, node_idx, zrows)
    t2, dinv = _tc_mid(p1p[0], p1p[1], r2(ddp[0]), r2(ddp[1]),
                       W1, b1.reshape(1, -1), ln1_g.reshape(1, -1),
                       ln1_b.reshape(1, -1), W2)
    a2p, = _sc_pass(t2, node_idx, edge_idx, zrows)
    a2s = _tc_combine2(a2p[0], a2p[1], bnorm)
    p2p, = _sc_pass(a2s, edge_idx, node_idx, zrows)
    out = _tc_final(p2p[0], p2p[1], dinv, b2.reshape(1, -1),
                    ln2_g.reshape(1, -1), ln2_b.reshape(1, -1), Wp,
                    bp.reshape(1, -1), lnp_g.reshape(1, -1),
                    lnp_b.reshape(1, -1))
    return out


# speedup vs baseline: 8.1038x; 1.0517x over previous
import functools

import jax
import jax.numpy as jnp
from jax import lax
from jax.experimental import pallas as pl
from jax.experimental.pallas import tpu as pltpu
from jax.experimental.pallas import tpu_sc as plsc

N = 10000
E = 320000
M = 10000
IN_DIM, HID, OUT_DIM, PROJ = 128, 256, 128, 128

NC, NS = 2, 16
WIN = 128
NWIN_T = 157
EPT = NWIN_T * WIN
E_PAD = NS * EPT
R_PAD = 10008
VPT = 1000

_MESH = plsc.VectorSubcoreMesh(core_axis_name="c", subcore_axis_name="s")


def _sc_pass_body(with_degrees, *refs):
    if with_degrees:
        (src, idx_s_hbm, idx_d_hbm, ew_hbm, zrows, out, dd_out, bd_out,
         idx_s, idx_d, rows0, rows1, w_v0, w_v1, w_v2, w_v3, one_v, zv_v,
         acc, dd_sh, bd_sh, g0, g1, s0, s1,
         wg0, wg1, wg2, wg3, d0, d1, d2, d3, bdsem) = refs
        w_vs = (w_v0, w_v1, w_v2, w_v3)
        wgs = (wg0, wg1, wg2, wg3)
        ds_ = (d0, d1, d2, d3)
    else:
        (src, idx_s_hbm, idx_d_hbm, zrows, out,
         idx_s, idx_d, rows0, rows1, acc, g0, g1, s0, s1) = refs
    rows = (rows0, rows1)
    gs = (g0, g1)
    ss = (s0, s1)
    c = lax.axis_index("c")
    s = lax.axis_index("s")

    @pl.when(s < 10)
    def _zero_acc():
        pltpu.sync_copy(zrows.at[pl.ds(s * VPT, VPT)],
                        acc.at[pl.ds(s * VPT, VPT)])
    pltpu.sync_copy(idx_s_hbm.at[s], idx_s)
    pltpu.sync_copy(idx_d_hbm.at[s], idx_d)
    if with_degrees:
        def _z(j, carry):
            zv_v[pl.ds(j * 16, 16)] = jnp.zeros((16,), jnp.float32)
            return carry
        lax.fori_loop(0, 63, _z, 0)
        @pl.when(s < 10)
        def _zero_deg():
            pltpu.sync_copy(zv_v.at[pl.ds(0, VPT)],
                            dd_sh.at[pl.ds(s * VPT, VPT)])
            pltpu.sync_copy(zv_v.at[pl.ds(0, VPT)],
                            bd_sh.at[pl.ds(s * VPT, VPT)])
        for j in range(WIN // 16):
            one_v[pl.ds(j * 16, 16)] = jnp.full((16,), 1.0, jnp.float32)
    plsc.subcore_barrier()

    def g_start(i, b):
        pltpu.async_copy(src.at[c].at[idx_s.at[i]], rows[b], gs[b])

    def g_wait(i, b):
        pltpu.make_async_copy(src.at[c].at[idx_s.at[i]], rows[b],
                              gs[b]).wait()

    def s_start(i, b):
        pltpu.async_copy(rows[b], acc.at[idx_d.at[i]], ss[b], add=True)

    def s_wait(i, b):
        pltpu.make_async_copy(rows[b], acc.at[idx_d.at[i]], ss[b]).wait()

    def wg_start(i, wb):
        @pl.when(c == 0)
        def _():
            pltpu.async_copy(ew_hbm.at[idx_d.at[i]], w_vs[wb], wgs[wb])

    def wg_wait(i, wb):
        @pl.when(c == 0)
        def _():
            pltpu.make_async_copy(ew_hbm.at[idx_d.at[i]], w_vs[wb],
                                  wgs[wb]).wait()

    def dd_start(i, wb):
        @pl.when(c == 0)
        def _():
            pltpu.async_copy(w_vs[wb], dd_sh.at[idx_s.at[i]], ds_[wb],
                             add=True)

    def dd_wait(i, wb):
        @pl.when(c == 0)
        def _():
            pltpu.make_async_copy(w_vs[wb], dd_sh.at[idx_s.at[i]],
                                  ds_[wb]).wait()

    def bd_start(i):
        @pl.when(c == 0)
        def _():
            pltpu.async_copy(one_v, bd_sh.at[idx_d.at[i]], bdsem, add=True)

    def bd_wait(i):
        @pl.when(c == 0)
        def _():
            pltpu.make_async_copy(one_v, bd_sh.at[idx_d.at[i]],
                                  bdsem).wait()

    g_start(0, 0)
    if with_degrees:
        wg_start(0, 0)
        wg_start(1, 1)

    def quad(j, carry):
        for b2 in range(4):
            i = 4 * j + b2
            b = b2 % 2
            ob = 1 - b
            g_wait(i, b)
            if with_degrees:
                nwb = (b2 + 2) % 4
                if b2 < 2:
                    @pl.when(j >= 1)
                    def _ddw(i=i, nwb=nwb):
                        dd_wait(i - 2, nwb)
                else:
                    dd_wait(i - 2, nwb)
                if b2 == 3:
                    @pl.when(i + 2 < NWIN_T)
                    def _wgs2(i=i, nwb=nwb):
                        wg_start(i + 2, nwb)
                else:
                    wg_start(i + 2, nwb)
                wg_wait(i, b2)
                dd_start(i, b2)
                bd_start(i)
            s_start(i, b)
            if b2 == 0:
                @pl.when(j >= 1)
                def _sw0(i=i, ob=ob):
                    s_wait(i - 1, ob)
            else:
                s_wait(i - 1, ob)
            g_start(i + 1, ob)
        return carry

    lax.fori_loop(0, NWIN_T // 4, quad, 0)
    li = NWIN_T - 1
    g_wait(li, 0)
    if with_degrees:
        wg_wait(li, 0)
        dd_start(li, 0)
        bd_start(li)
    s_start(li, 0)
    s_wait(li - 1, 1)
    s_wait(li, 0)
    if with_degrees:
        dd_wait(li - 2, 2)
        dd_wait(li - 1, 3)
        dd_wait(li, 0)
        def _bdrain(i, carry):
            bd_wait(i)
            return carry
        lax.fori_loop(0, NWIN_T, _bdrain, 0)
    plsc.subcore_barrier()
    @pl.when(s < 10)
    def _out_acc():
        pltpu.sync_copy(acc.at[pl.ds(s * VPT, VPT)],
                        out.at[c, pl.ds(s * VPT, VPT)])
    if with_degrees:
        @pl.when((s == 0) & (c == 0))
        def _out_deg():
            pltpu.sync_copy(dd_sh.at[pl.ds(0, N)], dd_out.at[0])
            pltpu.sync_copy(bd_sh.at[pl.ds(0, M)], bd_out.at[0])


def _make_sc_pass(with_degrees):
    out_type = [jax.ShapeDtypeStruct((NC, M, 64), jnp.float32)]
    scratch = [
        pltpu.VMEM((NWIN_T, WIN), jnp.int32),
        pltpu.VMEM((NWIN_T, WIN), jnp.int32),
        pltpu.VMEM((WIN, 64), jnp.float32),
        pltpu.VMEM((WIN, 64), jnp.float32),
    ]
    if with_degrees:
        out_type += [jax.ShapeDtypeStruct((1, N), jnp.float32),
                     jax.ShapeDtypeStruct((1, M), jnp.float32)]
        scratch += [pltpu.VMEM((WIN,), jnp.float32)] * 4
        scratch += [
            pltpu.VMEM((WIN,), jnp.float32),
            pltpu.VMEM((1008,), jnp.float32),
        ]
    scratch += [pltpu.VMEM_SHARED((R_PAD, 64), jnp.float32)]
    if with_degrees:
        scratch += [pltpu.VMEM_SHARED((R_PAD,), jnp.float32),
                    pltpu.VMEM_SHARED((R_PAD,), jnp.float32)]
    scratch += [pltpu.SemaphoreType.DMA] * (13 if with_degrees else 4)
    return pl.kernel(
        functools.partial(_sc_pass_body, with_degrees),
        out_type=out_type, mesh=_MESH, scratch_types=scratch,
        compiler_params=pltpu.CompilerParams(use_tc_tiling_on_sc=False),
        name="sc_pass_deg" if with_degrees else "sc_pass")


_sc_pass_deg = _make_sc_pass(True)
_sc_pass = _make_sc_pass(False)


_BR = 1000
_GRID = M // _BR


def _dot(a, b):
    return lax.dot_general(a, b, (((1,), (0,)), ((), ())),
                           preferred_element_type=jnp.float32,
                           precision=lax.Precision.HIGHEST)


def _ln(h, g, b, eps=1e-5):
    mu = jnp.mean(h, axis=-1, keepdims=True)
    var = jnp.mean((h - mu) ** 2, axis=-1, keepdims=True)
    return (h - mu) * lax.rsqrt(var + eps) * g + b


def _row_spec(w):
    return pl.BlockSpec((_BR, w), lambda i: (i, 0))


_SPLIT_OUT_SPEC = pl.BlockSpec((2, _BR, 64), lambda i: (0, i, 0))


def _full_spec(r, c):
    return pl.BlockSpec((r, c), lambda i: (0, 0))


def _cat(h0, h1):
    return jnp.concatenate([h0[...], h1[...]], axis=-1)


def _split_store(out, h):
    out[0] = h[:, :64]
    out[1] = h[:, 64:]


def _tc_combine_body(a0, a1, bd, ew, out_a, out_bn):
    b = bd[...]
    binv = jnp.where(b > 0, 1.0 / jnp.where(b > 0, b, 1.0), 0.0)
    bn = binv * ew[...]
    out_bn[...] = bn
    _split_store(out_a, _cat(a0, a1) * bn)


_tc_combine = pl.pallas_call(
    _tc_combine_body,
    grid=(_GRID,),
    in_specs=[_row_spec(64), _row_spec(64), _row_spec(1), _row_spec(1)],
    out_specs=[_SPLIT_OUT_SPEC, _row_spec(1)],
    out_shape=[jax.ShapeDtypeStruct((2, R_PAD, 64), jnp.float32),
               jax.ShapeDtypeStruct((M, 1), jnp.float32)],
)


def _tc_combine2_body(a0, a1, bn, out_a):
    _split_store(out_a, _cat(a0, a1) * bn[...])


_tc_combine2 = pl.pallas_call(
    _tc_combine2_body,
    grid=(_GRID,),
    in_specs=[_row_spec(64), _row_spec(64), _row_spec(1)],
    out_specs=_SPLIT_OUT_SPEC,
    out_shape=jax.ShapeDtypeStruct((2, R_PAD, 64), jnp.float32),
)


def _tc_mid_body(p0, p1, dd, W1, b1, g1, bb1, W2, out_t2, out_dinv):
    d = dd[...]
    dinv = jnp.where(d > 0, 1.0 / jnp.where(d > 0, d, 1.0), 0.0)
    out_dinv[...] = dinv
    p = _cat(p0, p1) * dinv
    h = _dot(p, W1[...]) + b1[...]
    h = jnp.maximum(_ln(h, g1[...], bb1[...]), 0.0)
    _split_store(out_t2, _dot(h, W2[...]))


_tc_mid = pl.pallas_call(
    _tc_mid_body,
    grid=(_GRID,),
    in_specs=[_row_spec(64), _row_spec(64), _row_spec(1),
              _full_spec(128, HID), _full_spec(1, HID), _full_spec(1, HID),
              _full_spec(1, HID), _full_spec(HID, 128)],
    out_specs=[_SPLIT_OUT_SPEC, _row_spec(1)],
    out_shape=[jax.ShapeDtypeStruct((2, R_PAD, 64), jnp.float32),
               jax.ShapeDtypeStruct((N, 1), jnp.float32)],
)


def _tc_final_body(p0, p1, dinv, b2, g2, bb2, Wp, bp, gp, bbp, out):
    p = _cat(p0, p1) * dinv[...] + b2[...]
    h = _ln(p, g2[...], bb2[...])
    h = jnp.maximum(_dot(h, Wp[...]) + bp[...], 0.0)
    out[...] = _ln(h, gp[...], bbp[...])


_tc_final = pl.pallas_call(
    _tc_final_body,
    grid=(_GRID,),
    in_specs=[_row_spec(64), _row_spec(64), _row_spec(1),
              _full_spec(1, 128), _full_spec(1, 128), _full_spec(1, 128),
              _full_spec(128, 128), _full_spec(1, 128), _full_spec(1, 128),
              _full_spec(1, 128)],
    out_specs=_row_spec(128),
    out_shape=jax.ShapeDtypeStruct((N, PROJ), jnp.float32),
)


def kernel(x, edge_index, edge_weight, W1, b1, ln1_g, ln1_b, W2, b2, ln2_g,
           ln2_b, Wp, bp, lnp_g, lnp_b):
    pad = E_PAD - E
    node_idx = jnp.concatenate(
        [edge_index[0], jnp.full((pad,), N, jnp.int32)]).reshape(
            NS, NWIN_T, WIN)
    edge_idx = jnp.concatenate(
        [edge_index[1], jnp.full((pad,), M, jnp.int32)]).reshape(
            NS, NWIN_T, WIN)
    ew_pad = jnp.concatenate([edge_weight, jnp.zeros((8,), jnp.float32)])
    zrows = jnp.zeros((R_PAD, 64), jnp.float32)
    xs = jnp.zeros((2, R_PAD, 64), jnp.float32)
    xs = xs.at[0, :N].set(x[:, :64]).at[1, :N].set(x[:, 64:])

    r2 = lambda v: v.reshape(-1, 1)
    a1p, ddp, bdp = _sc_pass_deg(xs, node_idx, edge_idx, ew_pad, zrows)
    a1s, bnorm = _tc_combine(a1p[0], a1p[1], r2(bdp), r2(edge_weight))
    p1p, = _sc_pass(a1s, edge_idx, node_idx, zrows)
    t2, dinv = _tc_mid(p1p[0], p1p[1], r2(ddp),
                       W1, b1.reshape(1, -1), ln1_g.reshape(1, -1),
                       ln1_b.reshape(1, -1), W2)
    a2p, = _sc_pass(t2, node_idx, edge_idx, zrows)
    a2s = _tc_combine2(a2p[0], a2p[1], bnorm)
    p2p, = _sc_pass(a2s, edge_idx, node_idx, zrows)
    out = _tc_final(p2p[0], p2p[1], dinv, b2.reshape(1, -1),
                    ln2_g.reshape(1, -1), ln2_b.reshape(1, -1), Wp,
                    bp.reshape(1, -1), lnp_g.reshape(1, -1),
                    lnp_b.reshape(1, -1))
    return out


# speedup vs baseline: 10.6424x; 1.3133x over previous
import functools

import jax
import jax.numpy as jnp
from jax import lax
from jax.experimental import pallas as pl
from jax.experimental.pallas import tpu as pltpu
from jax.experimental.pallas import tpu_sc as plsc

N = 10000
E = 320000
M = 10000
IN_DIM, HID, OUT_DIM, PROJ = 128, 256, 128, 128

NC, NS = 2, 16
WIN = 128
NWIN_T = 157
EPT = NWIN_T * WIN
E_PAD = NS * EPT
R_PAD = 10008
VPT = 1000

_MESH = plsc.VectorSubcoreMesh(core_axis_name="c", subcore_axis_name="s")


def _sc_pass_body(with_degrees, *refs):
    if with_degrees:
        (src, idx_s_hbm, idx_d_hbm, ew_hbm, zrows, out, dd_out, bd_out,
         idx_s, idx_d, rows0, rows1, rows2, rows3,
         w_v0, w_v1, w_v2, w_v3, one_v, zv_v,
         acc, dd_sh, bd_sh, g0, g1, g2, g3, s0, s1, s2, s3,
         wg0, wg1, wg2, wg3, d0, d1, d2, d3, bdsem) = refs
        w_vs = (w_v0, w_v1, w_v2, w_v3)
        wgs = (wg0, wg1, wg2, wg3)
        ds_ = (d0, d1, d2, d3)
    else:
        (src, idx_s_hbm, idx_d_hbm, zrows, out,
         idx_s, idx_d, rows0, rows1, rows2, rows3, acc,
         g0, g1, g2, g3, s0, s1, s2, s3) = refs
    rows = (rows0, rows1, rows2, rows3)
    gs = (g0, g1, g2, g3)
    ss = (s0, s1, s2, s3)
    c = lax.axis_index("c")
    s = lax.axis_index("s")

    @pl.when(s < 10)
    def _zero_acc():
        pltpu.sync_copy(zrows.at[pl.ds(s * VPT, VPT)],
                        acc.at[pl.ds(s * VPT, VPT)])
    pltpu.sync_copy(idx_s_hbm.at[s], idx_s)
    pltpu.sync_copy(idx_d_hbm.at[s], idx_d)
    if with_degrees:
        def _z(j, carry):
            zv_v[pl.ds(j * 16, 16)] = jnp.zeros((16,), jnp.float32)
            return carry
        lax.fori_loop(0, 63, _z, 0)
        @pl.when(s < 10)
        def _zero_deg():
            pltpu.sync_copy(zv_v.at[pl.ds(0, VPT)],
                            dd_sh.at[pl.ds(s * VPT, VPT)])
            pltpu.sync_copy(zv_v.at[pl.ds(0, VPT)],
                            bd_sh.at[pl.ds(s * VPT, VPT)])
        for j in range(WIN // 16):
            one_v[pl.ds(j * 16, 16)] = jnp.full((16,), 1.0, jnp.float32)
    plsc.subcore_barrier()

    def g_start(i, b):
        pltpu.async_copy(src.at[c].at[idx_s.at[i]], rows[b], gs[b])

    def g_wait(i, b):
        pltpu.make_async_copy(src.at[c].at[idx_s.at[i]], rows[b],
                              gs[b]).wait()

    def s_start(i, b):
        pltpu.async_copy(rows[b], acc.at[idx_d.at[i]], ss[b], add=True)

    def s_wait(i, b):
        pltpu.make_async_copy(rows[b], acc.at[idx_d.at[i]], ss[b]).wait()

    def wg_start(i, wb):
        @pl.when(c == 0)
        def _():
            pltpu.async_copy(ew_hbm.at[idx_d.at[i]], w_vs[wb], wgs[wb])

    def wg_wait(i, wb):
        @pl.when(c == 0)
        def _():
            pltpu.make_async_copy(ew_hbm.at[idx_d.at[i]], w_vs[wb],
                                  wgs[wb]).wait()

    def dd_start(i, wb):
        @pl.when(c == 0)
        def _():
            pltpu.async_copy(w_vs[wb], dd_sh.at[idx_s.at[i]], ds_[wb],
                             add=True)

    def dd_wait(i, wb):
        @pl.when(c == 0)
        def _():
            pltpu.make_async_copy(w_vs[wb], dd_sh.at[idx_s.at[i]],
                                  ds_[wb]).wait()

    def bd_start(i):
        @pl.when(c == 0)
        def _():
            pltpu.async_copy(one_v, bd_sh.at[idx_d.at[i]], bdsem, add=True)

    def bd_wait(i):
        @pl.when(c == 0)
        def _():
            pltpu.make_async_copy(one_v, bd_sh.at[idx_d.at[i]],
                                  bdsem).wait()

    g_start(0, 0)
    g_start(1, 1)
    if with_degrees:
        wg_start(0, 0)
        wg_start(1, 1)

    def quad(j, carry):
        for b2 in range(4):
            i = 4 * j + b2
            nb = (b2 + 2) % 4
            if b2 < 2:
                @pl.when(j >= 1)
                def _sw(i=i, nb=nb):
                    s_wait(i - 2, nb)
            else:
                s_wait(i - 2, nb)
            if b2 == 3:
                @pl.when(i + 2 < NWIN_T)
                def _gs(i=i, nb=nb):
                    g_start(i + 2, nb)
            else:
                g_start(i + 2, nb)
            if with_degrees:
                if b2 < 2:
                    @pl.when(j >= 1)
                    def _ddw(i=i, nb=nb):
                        dd_wait(i - 2, nb)
                else:
                    dd_wait(i - 2, nb)
                if b2 == 3:
                    @pl.when(i + 2 < NWIN_T)
                    def _wgs2(i=i, nb=nb):
                        wg_start(i + 2, nb)
                else:
                    wg_start(i + 2, nb)
            g_wait(i, b2)
            if with_degrees:
                wg_wait(i, b2)
                dd_start(i, b2)
                bd_start(i)
            s_start(i, b2)
        return carry

    lax.fori_loop(0, NWIN_T // 4, quad, 0)
    li = NWIN_T - 1
    g_wait(li, 0)
    if with_degrees:
        wg_wait(li, 0)
        dd_start(li, 0)
        bd_start(li)
    s_start(li, 0)
    s_wait(li - 2, 2)
    s_wait(li - 1, 3)
    s_wait(li, 0)
    if with_degrees:
        dd_wait(li - 2, 2)
        dd_wait(li - 1, 3)
        dd_wait(li, 0)
        def _bdrain(i, carry):
            bd_wait(i)
            return carry
        lax.fori_loop(0, NWIN_T, _bdrain, 0)
    plsc.subcore_barrier()
    @pl.when(s < 10)
    def _out_acc():
        pltpu.sync_copy(acc.at[pl.ds(s * VPT, VPT)],
                        out.at[c, pl.ds(s * VPT, VPT)])
    if with_degrees:
        @pl.when((s == 0) & (c == 0))
        def _out_deg():
            pltpu.sync_copy(dd_sh.at[pl.ds(0, N)], dd_out.at[0])
            pltpu.sync_copy(bd_sh.at[pl.ds(0, M)], bd_out.at[0])


def _make_sc_pass(with_degrees):
    out_type = [jax.ShapeDtypeStruct((NC, M, 64), jnp.float32)]
    scratch = [
        pltpu.VMEM((NWIN_T, WIN), jnp.int32),
        pltpu.VMEM((NWIN_T, WIN), jnp.int32),
    ]
    scratch += [pltpu.VMEM((WIN, 64), jnp.float32)] * 4
    if with_degrees:
        out_type += [jax.ShapeDtypeStruct((1, N), jnp.float32),
                     jax.ShapeDtypeStruct((1, M), jnp.float32)]
        scratch += [pltpu.VMEM((WIN,), jnp.float32)] * 4
        scratch += [
            pltpu.VMEM((WIN,), jnp.float32),
            pltpu.VMEM((1008,), jnp.float32),
        ]
    scratch += [pltpu.VMEM_SHARED((R_PAD, 64), jnp.float32)]
    if with_degrees:
        scratch += [pltpu.VMEM_SHARED((R_PAD,), jnp.float32),
                    pltpu.VMEM_SHARED((R_PAD,), jnp.float32)]
    scratch += [pltpu.SemaphoreType.DMA] * (17 if with_degrees else 8)
    return pl.kernel(
        functools.partial(_sc_pass_body, with_degrees),
        out_type=out_type, mesh=_MESH, scratch_types=scratch,
        compiler_params=pltpu.CompilerParams(use_tc_tiling_on_sc=False),
        name="sc_pass_deg" if with_degrees else "sc_pass")


_sc_pass_deg = _make_sc_pass(True)
_sc_pass = _make_sc_pass(False)


_BR = 1000
_GRID = M // _BR


def _dot(a, b):
    return lax.dot_general(a, b, (((1,), (0,)), ((), ())),
                           preferred_element_type=jnp.float32,
                           precision=lax.Precision.HIGHEST)


def _ln(h, g, b, eps=1e-5):
    mu = jnp.mean(h, axis=-1, keepdims=True)
    var = jnp.mean((h - mu) ** 2, axis=-1, keepdims=True)
    return (h - mu) * lax.rsqrt(var + eps) * g + b


def _row_spec(w):
    return pl.BlockSpec((_BR, w), lambda i: (i, 0))


_SPLIT_OUT_SPEC = pl.BlockSpec((2, _BR, 64), lambda i: (0, i, 0))


def _full_spec(r, c):
    return pl.BlockSpec((r, c), lambda i: (0, 0))


def _cat(h0, h1):
    return jnp.concatenate([h0[...], h1[...]], axis=-1)


def _split_store(out, h):
    out[0] = h[:, :64]
    out[1] = h[:, 64:]


def _tc_combine_body(a0, a1, bd, ew, out_a, out_bn):
    b = bd[...]
    binv = jnp.where(b > 0, 1.0 / jnp.where(b > 0, b, 1.0), 0.0)
    bn = binv * ew[...]
    out_bn[...] = bn
    _split_store(out_a, _cat(a0, a1) * bn)


_tc_combine = pl.pallas_call(
    _tc_combine_body,
    grid=(_GRID,),
    in_specs=[_row_spec(64), _row_spec(64), _row_spec(1), _row_spec(1)],
    out_specs=[_SPLIT_OUT_SPEC, _row_spec(1)],
    out_shape=[jax.ShapeDtypeStruct((2, R_PAD, 64), jnp.float32),
               jax.ShapeDtypeStruct((M, 1), jnp.float32)],
)


def _tc_combine2_body(a0, a1, bn, out_a):
    _split_store(out_a, _cat(a0, a1) * bn[...])


_tc_combine2 = pl.pallas_call(
    _tc_combine2_body,
    grid=(_GRID,),
    in_specs=[_row_spec(64), _row_spec(64), _row_spec(1)],
    out_specs=_SPLIT_OUT_SPEC,
    out_shape=jax.ShapeDtypeStruct((2, R_PAD, 64), jnp.float32),
)


def _tc_mid_body(p0, p1, dd, W1, b1, g1, bb1, W2, out_t2, out_dinv):
    d = dd[...]
    dinv = jnp.where(d > 0, 1.0 / jnp.where(d > 0, d, 1.0), 0.0)
    out_dinv[...] = dinv
    p = _cat(p0, p1) * dinv
    h = _dot(p, W1[...]) + b1[...]
    h = jnp.maximum(_ln(h, g1[...], bb1[...]), 0.0)
    _split_store(out_t2, _dot(h, W2[...]))


_tc_mid = pl.pallas_call(
    _tc_mid_body,
    grid=(_GRID,),
    in_specs=[_row_spec(64), _row_spec(64), _row_spec(1),
              _full_spec(128, HID), _full_spec(1, HID), _full_spec(1, HID),
              _full_spec(1, HID), _full_spec(HID, 128)],
    out_specs=[_SPLIT_OUT_SPEC, _row_spec(1)],
    out_shape=[jax.ShapeDtypeStruct((2, R_PAD, 64), jnp.float32),
               jax.ShapeDtypeStruct((N, 1), jnp.float32)],
)


def _tc_final_body(p0, p1, dinv, b2, g2, bb2, Wp, bp, gp, bbp, out):
    p = _cat(p0, p1) * dinv[...] + b2[...]
    h = _ln(p, g2[...], bb2[...])
    h = jnp.maximum(_dot(h, Wp[...]) + bp[...], 0.0)
    out[...] = _ln(h, gp[...], bbp[...])


_tc_final = pl.pallas_call(
    _tc_final_body,
    grid=(_GRID,),
    in_specs=[_row_spec(64), _row_spec(64), _row_spec(1),
              _full_spec(1, 128), _full_spec(1, 128), _full_spec(1, 128),
              _full_spec(128, 128), _full_spec(1, 128), _full_spec(1, 128),
              _full_spec(1, 128)],
    out_specs=_row_spec(128),
    out_shape=jax.ShapeDtypeStruct((N, PROJ), jnp.float32),
)


def kernel(x, edge_index, edge_weight, W1, b1, ln1_g, ln1_b, W2, b2, ln2_g,
           ln2_b, Wp, bp, lnp_g, lnp_b):
    pad = E_PAD - E
    node_idx = jnp.concatenate(
        [edge_index[0], jnp.full((pad,), N, jnp.int32)]).reshape(
            NS, NWIN_T, WIN)
    edge_idx = jnp.concatenate(
        [edge_index[1], jnp.full((pad,), M, jnp.int32)]).reshape(
            NS, NWIN_T, WIN)
    ew_pad = jnp.concatenate([edge_weight, jnp.zeros((8,), jnp.float32)])
    zrows = jnp.zeros((R_PAD, 64), jnp.float32)
    xs = jnp.zeros((2, R_PAD, 64), jnp.float32)
    xs = xs.at[0, :N].set(x[:, :64]).at[1, :N].set(x[:, 64:])

    r2 = lambda v: v.reshape(-1, 1)
    a1p, ddp, bdp = _sc_pass_deg(xs, node_idx, edge_idx, ew_pad, zrows)
    a1s, bnorm = _tc_combine(a1p[0], a1p[1], r2(bdp), r2(edge_weight))
    p1p, = _sc_pass(a1s, edge_idx, node_idx, zrows)
    t2, dinv = _tc_mid(p1p[0], p1p[1], r2(ddp),
                       W1, b1.reshape(1, -1), ln1_g.reshape(1, -1),
                       ln1_b.reshape(1, -1), W2)
    a2p, = _sc_pass(t2, node_idx, edge_idx, zrows)
    a2s = _tc_combine2(a2p[0], a2p[1], bnorm)
    p2p, = _sc_pass(a2s, edge_idx, node_idx, zrows)
    out = _tc_final(p2p[0], p2p[1], dinv, b2.reshape(1, -1),
                    ln2_g.reshape(1, -1), ln2_b.reshape(1, -1), Wp,
                    bp.reshape(1, -1), lnp_g.reshape(1, -1),
                    lnp_b.reshape(1, -1))
    return out
